# baseline (speedup 1.0000x reference)
import functools

import jax
import jax.numpy as jnp
from jax import lax
from jax.experimental import pallas as pl
from jax.experimental.pallas import tpu as pltpu
from jax.experimental.pallas import tpu_sc as plsc

_B = 4096
_L = 200
_D = 64
_V = 1000
_info = plsc.get_sparse_core_info()
_NC = _info.num_cores
_NS = _info.num_subcores
_NW = _NC * _NS
_BPW = _B // _NW
_UNIT = _BPW * _D
_CHUNK = 8 * _BPW
_NBUF = 4
_NGRP = _L // _NBUF
_QS = 4
_QROWS = _BPW // _QS

_mesh = plsc.VectorSubcoreMesh(core_axis_name="c", subcore_axis_name="s")


@functools.partial(
    pl.kernel,
    mesh=_mesh,
    out_type=jax.ShapeDtypeStruct((_L, _NW * _UNIT), jnp.float32),
    scratch_types=[
        pltpu.VMEM((_V * _D,), jnp.float32),
        pltpu.VMEM((_QROWS * _L,), jnp.int32),
        pltpu.VMEM((_L, _BPW), jnp.int32),
        [pltpu.VMEM((_UNIT,), jnp.float32)] * _NBUF,
        pltpu.SemaphoreType.DMA,
        pltpu.SemaphoreType.DMA((_NBUF,)),
    ],
    compiler_params=pltpu.CompilerParams(
        use_tc_tiling_on_sc=False, needs_layout_passes=False),
)
def _sc_embed(idx_hbm, table_hbm, out_hbm, tab_v, ibuf, bases_v, unit_v,
              tsem, osem):
    wid = lax.axis_index("s") * _NC + lax.axis_index("c")

    pltpu.async_copy(table_hbm, tab_v, tsem)

    lane = lax.iota(jnp.int32, 16)
    for q in range(_QS):
        pltpu.sync_copy(
            idx_hbm.at[pl.ds((wid * _BPW + q * _QROWS) * _L, _QROWS * _L)],
            ibuf)

        def tr_body(l, carry):
            for g in range(_QROWS // 16):
                addr = (lane + g * 16) * _L + l
                v = plsc.load_gather(ibuf, [addr])
                bases_v[l, pl.ds(q * _QROWS + g * 16, 16)] = v * _D
            return carry

        lax.fori_loop(0, _L, tr_body, 0)

    pltpu.make_async_copy(table_hbm, tab_v, tsem).wait()

    def fill_unit(l, b):
        base = [bases_v[l, pl.ds(g * 16, 16)] for g in range(_BPW // 16)]

        def s_body(s, carry):
            rot = (lane + s) & 15
            rotst = rot * _BPW + lane
            for g in range(_BPW // 16):
                bg = base[g] + rot
                for dq in range(_D // 16):
                    v = plsc.load_gather(tab_v, [bg + dq * 16])
                    plsc.store_scatter(
                        unit_v[b], [rotst + (dq * 16 * _BPW + g * 16)], v)
            return carry

        lax.fori_loop(0, 16, s_body, 0)

    def start_out(l, b):
        for dh in range(_D // 8):
            pltpu.async_copy(
                unit_v[b].at[pl.ds(dh * _CHUNK, _CHUNK)],
                out_hbm.at[l, pl.ds(dh * _NW * _CHUNK + wid * _CHUNK,
                                    _CHUNK)],
                osem.at[b])

    def wait_out(b):
        pltpu.make_async_copy(
            out_hbm.at[0, pl.ds(0, _UNIT)], unit_v[b], osem.at[b]).wait()

    for b in range(_NBUF):
        fill_unit(b, b)
        start_out(b, b)

    def group(g, carry):
        for b in range(_NBUF):
            l = g * _NBUF + b
            wait_out(b)
            fill_unit(l, b)
            start_out(l, b)
        return carry

    lax.fori_loop(1, _NGRP, group, 0)

    for b in range(_NBUF):
        wait_out(b)


def kernel(visit_segments, embedding_table):
    p = _sc_embed(visit_segments.reshape(-1), embedding_table.reshape(-1))
    p = p.reshape(_L, _D // 8, _NW, 8, _BPW)
    return p.transpose(2, 4, 0, 1, 3).reshape(_B, _L, _D)


# speedup vs baseline: 2.1989x; 2.1989x over previous
import functools

import jax
import jax.numpy as jnp
from jax import lax
from jax.experimental import pallas as pl
from jax.experimental.pallas import tpu as pltpu
from jax.experimental.pallas import tpu_sc as plsc

_B = 4096
_L = 200
_D = 64
_V = 1000
_info = plsc.get_sparse_core_info()
_NC = _info.num_cores
_NS = _info.num_subcores
_NW = _NC * _NS
_BPW = _B // _NW
_UNIT = _BPW * _D
_CHUNK = 8 * _BPW
_NBUF = 4
_NGRP = _L // _NBUF
_QS = 4
_QROWS = _BPW // _QS

_mesh = plsc.VectorSubcoreMesh(core_axis_name="c", subcore_axis_name="s")


@functools.partial(
    pl.kernel,
    mesh=_mesh,
    out_type=jax.ShapeDtypeStruct((_L, _NW * _UNIT), jnp.float32),
    scratch_types=[
        pltpu.VMEM((_V * _D,), jnp.float32),
        pltpu.VMEM((_QROWS * _L,), jnp.int32),
        pltpu.VMEM((_L, _BPW), jnp.int32),
        [pltpu.VMEM((_UNIT,), jnp.float32)] * _NBUF,
        pltpu.SemaphoreType.DMA,
        pltpu.SemaphoreType.DMA((_NBUF,)),
    ],
    compiler_params=pltpu.CompilerParams(
        use_tc_tiling_on_sc=False, needs_layout_passes=False),
)
def _sc_embed(idx_hbm, table_hbm, out_hbm, tab_v, ibuf, bases_v, unit_v,
              tsem, osem):
    wid = lax.axis_index("s") * _NC + lax.axis_index("c")

    pltpu.async_copy(table_hbm, tab_v, tsem)

    lane = lax.iota(jnp.int32, 16)
    for q in range(_QS):
        pltpu.sync_copy(
            idx_hbm.at[pl.ds((wid * _BPW + q * _QROWS) * _L, _QROWS * _L)],
            ibuf)

        def tr_body(l, carry):
            for g in range(_QROWS // 16):
                addr = (lane + g * 16) * _L + l
                v = plsc.load_gather(ibuf, [addr])
                bases_v[l, pl.ds(q * _QROWS + g * 16, 16)] = v * _D
            return carry

        lax.fori_loop(0, _L, tr_body, 0)

    pltpu.make_async_copy(table_hbm, tab_v, tsem).wait()

    def fill_unit(l, b):
        base = [bases_v[l, pl.ds(g * 16, 16)] for g in range(_BPW // 16)]

        def s_body(s, carry):
            rot = (lane + s) & 15
            rotst = rot * _BPW + lane
            vals = []
            for g in range(_BPW // 16):
                bg = base[g] + rot
                for dq in range(_D // 16):
                    vals.append((plsc.load_gather(tab_v, [bg + dq * 16]),
                                 dq * 16 * _BPW + g * 16))
            for v, off in vals:
                plsc.store_scatter(unit_v[b], [rotst + off], v)
            return carry

        lax.fori_loop(0, 16, s_body, 0)

    def start_out(l, b):
        for dh in range(_D // 8):
            pltpu.async_copy(
                unit_v[b].at[pl.ds(dh * _CHUNK, _CHUNK)],
                out_hbm.at[l, pl.ds(dh * _NW * _CHUNK + wid * _CHUNK,
                                    _CHUNK)],
                osem.at[b])

    def wait_out(b):
        pltpu.make_async_copy(
            out_hbm.at[0, pl.ds(0, _UNIT)], unit_v[b], osem.at[b]).wait()

    for b in range(_NBUF):
        fill_unit(b, b)
        start_out(b, b)

    def group(g, carry):
        for b in range(_NBUF):
            l = g * _NBUF + b
            wait_out(b)
            fill_unit(l, b)
            start_out(l, b)
        return carry

    lax.fori_loop(1, _NGRP, group, 0)

    for b in range(_NBUF):
        wait_out(b)


def kernel(visit_segments, embedding_table):
    p = _sc_embed(visit_segments.reshape(-1), embedding_table.reshape(-1))
    p = p.reshape(_L, _D // 8, _NW, 8, _BPW)
    return p.transpose(2, 4, 0, 1, 3).reshape(_B, _L, _D)


# speedup vs baseline: 2.2330x; 1.0155x over previous
import functools

import jax
import jax.numpy as jnp
from jax import lax
from jax.experimental import pallas as pl
from jax.experimental.pallas import tpu as pltpu
from jax.experimental.pallas import tpu_sc as plsc

_B = 4096
_L = 200
_D = 64
_V = 1000
_info = plsc.get_sparse_core_info()
_NC = _info.num_cores
_NS = _info.num_subcores
_NW = _NC * _NS
_BPW = _B // _NW
_UNIT = _BPW * _D
_CHUNK = 8 * _BPW
_NBUF = 4
_NGRP = _L // _NBUF
_QS = 4
_QROWS = _BPW // _QS

_mesh = plsc.VectorSubcoreMesh(core_axis_name="c", subcore_axis_name="s")


@functools.partial(
    pl.kernel,
    mesh=_mesh,
    out_type=jax.ShapeDtypeStruct((_L, _NW * _UNIT), jnp.float32),
    scratch_types=[
        pltpu.VMEM((_V * _D,), jnp.float32),
        pltpu.VMEM((_QROWS * _L,), jnp.int32),
        pltpu.VMEM((_L, _BPW), jnp.int32),
        [pltpu.VMEM((_UNIT,), jnp.float32)] * _NBUF,
        pltpu.SemaphoreType.DMA,
        pltpu.SemaphoreType.DMA((_NBUF,)),
    ],
    compiler_params=pltpu.CompilerParams(
        use_tc_tiling_on_sc=False, needs_layout_passes=False),
)
def _sc_embed(idx_hbm, table_hbm, out_hbm, tab_v, ibuf, bases_v, unit_v,
              tsem, osem):
    wid = lax.axis_index("s") * _NC + lax.axis_index("c")

    pltpu.async_copy(table_hbm, tab_v, tsem)

    lane = lax.iota(jnp.int32, 16)
    for q in range(_QS):
        pltpu.sync_copy(
            idx_hbm.at[pl.ds((wid * _BPW + q * _QROWS) * _L, _QROWS * _L)],
            ibuf)

        def tr_body(lb, carry):
            vals = []
            for dl in range(8):
                l = lb * 8 + dl
                for g in range(_QROWS // 16):
                    addr = (lane + g * 16) * _L + l
                    vals.append((l, g, plsc.load_gather(ibuf, [addr])))
            for l, g, v in vals:
                bases_v[l, pl.ds(q * _QROWS + g * 16, 16)] = v * _D
            return carry

        lax.fori_loop(0, _L // 8, tr_body, 0)

    pltpu.make_async_copy(table_hbm, tab_v, tsem).wait()

    def fill_unit(l, b):
        base = [bases_v[l, pl.ds(g * 16, 16)] for g in range(_BPW // 16)]

        def s_body(s, carry):
            rot = (lane + s) & 15
            rotst = rot * _BPW + lane
            vals = []
            for g in range(_BPW // 16):
                bg = base[g] + rot
                for dq in range(_D // 16):
                    vals.append((plsc.load_gather(tab_v, [bg + dq * 16]),
                                 dq * 16 * _BPW + g * 16))
            for v, off in vals:
                plsc.store_scatter(unit_v[b], [rotst + off], v)
            return carry

        lax.fori_loop(0, 16, s_body, 0)

    def start_out(l, b):
        for dh in range(_D // 8):
            pltpu.async_copy(
                unit_v[b].at[pl.ds(dh * _CHUNK, _CHUNK)],
                out_hbm.at[l, pl.ds(dh * _NW * _CHUNK + wid * _CHUNK,
                                    _CHUNK)],
                osem.at[b])

    def wait_out(b):
        pltpu.make_async_copy(
            out_hbm.at[0, pl.ds(0, _UNIT)], unit_v[b], osem.at[b]).wait()

    for b in range(_NBUF):
        fill_unit(b, b)
        start_out(b, b)

    def group(g, carry):
        for b in range(_NBUF):
            l = g * _NBUF + b
            wait_out(b)
            fill_unit(l, b)
            start_out(l, b)
        return carry

    lax.fori_loop(1, _NGRP, group, 0)

    for b in range(_NBUF):
        wait_out(b)


def kernel(visit_segments, embedding_table):
    p = _sc_embed(visit_segments.reshape(-1), embedding_table.reshape(-1))
    p = p.reshape(_L, _D // 8, _NW, 8, _BPW)
    return p.transpose(2, 4, 0, 1, 3).reshape(_B, _L, _D)


# speedup vs baseline: 2.4339x; 1.0900x over previous
import functools

import jax
import jax.numpy as jnp
from jax import lax
from jax.experimental import pallas as pl
from jax.experimental.pallas import tpu as pltpu
from jax.experimental.pallas import tpu_sc as plsc

_B = 4096
_L = 200
_D = 64
_V = 1000
_info = plsc.get_sparse_core_info()
_NC = _info.num_cores
_NS = _info.num_subcores
_NW = _NC * _NS
_BPW = _B // _NW
_UNIT = _BPW * _D
_CHUNK = 8 * _BPW
_NBUF = 4
_NGRP = _L // _NBUF
_QS = 4
_QROWS = _BPW // _QS

_mesh = plsc.VectorSubcoreMesh(core_axis_name="c", subcore_axis_name="s")


@functools.partial(
    pl.kernel,
    mesh=_mesh,
    out_type=jax.ShapeDtypeStruct((_L, _NW * _UNIT), jnp.float32),
    scratch_types=[
        pltpu.VMEM((_V * _D,), jnp.float32),
        pltpu.VMEM((_QROWS * _L,), jnp.int32),
        pltpu.VMEM((_L, _BPW), jnp.int32),
        [pltpu.VMEM((_UNIT,), jnp.float32)] * _NBUF,
        pltpu.SemaphoreType.DMA,
        pltpu.SemaphoreType.DMA((_NBUF,)),
    ],
    compiler_params=pltpu.CompilerParams(
        use_tc_tiling_on_sc=False, needs_layout_passes=False),
)
def _sc_embed(idx_hbm, table_hbm, out_hbm, tab_v, ibuf, bases_v, unit_v,
              tsem, osem):
    wid = lax.axis_index("s") * _NC + lax.axis_index("c")

    pltpu.async_copy(table_hbm, tab_v, tsem)

    lane = lax.iota(jnp.int32, 16)
    for q in range(_QS):
        pltpu.sync_copy(
            idx_hbm.at[pl.ds((wid * _BPW + q * _QROWS) * _L, _QROWS * _L)],
            ibuf)

        def tr_body(lb, carry):
            vals = []
            for dl in range(8):
                l = lb * 8 + dl
                for g in range(_QROWS // 16):
                    addr = (lane + g * 16) * _L + l
                    vals.append((l, g, plsc.load_gather(ibuf, [addr])))
            for l, g, v in vals:
                bases_v[l, pl.ds(q * _QROWS + g * 16, 16)] = v * _D
            return carry

        lax.fori_loop(0, _L // 8, tr_body, 0)

    pltpu.make_async_copy(table_hbm, tab_v, tsem).wait()

    def fill_unit(l, b):
        base = [bases_v[l, pl.ds(g * 16, 16)] for g in range(_BPW // 16)]

        def halfpairs(h):
            return [(g, dq) for g in range(h * 4, h * 4 + 4)
                    for dq in range(_D // 16)]

        def load_half(rot, h):
            return tuple(
                plsc.load_gather(tab_v, [base[g] + rot + dq * 16])
                for g, dq in halfpairs(h))

        def store_half(rotst, h, vals):
            for (g, dq), v in zip(halfpairs(h), vals):
                plsc.store_scatter(
                    unit_v[b], [rotst + (dq * 16 * _BPW + g * 16)], v)

        rot15 = (lane + 15) & 15
        rotst15 = rot15 * _BPW + lane
        carry0 = (load_half(rot15, 1), rotst15)

        def s_body(s, carry):
            pvals, protst = carry
            rot = (lane + s) & 15
            rotst = rot * _BPW + lane
            v0 = load_half(rot, 0)
            store_half(protst, 1, pvals)
            v1 = load_half(rot, 1)
            store_half(rotst, 0, v0)
            return (v1, rotst)

        vlast, rotstlast = lax.fori_loop(0, 16, s_body, carry0)
        store_half(rotstlast, 1, vlast)

    def start_out(l, b):
        for dh in range(_D // 8):
            pltpu.async_copy(
                unit_v[b].at[pl.ds(dh * _CHUNK, _CHUNK)],
                out_hbm.at[l, pl.ds(dh * _NW * _CHUNK + wid * _CHUNK,
                                    _CHUNK)],
                osem.at[b])

    def wait_out(b):
        pltpu.make_async_copy(
            out_hbm.at[0, pl.ds(0, _UNIT)], unit_v[b], osem.at[b]).wait()

    for b in range(_NBUF):
        fill_unit(b, b)
        start_out(b, b)

    def group(g, carry):
        for b in range(_NBUF):
            l = g * _NBUF + b
            wait_out(b)
            fill_unit(l, b)
            start_out(l, b)
        return carry

    lax.fori_loop(1, _NGRP, group, 0)

    for b in range(_NBUF):
        wait_out(b)


def kernel(visit_segments, embedding_table):
    p = _sc_embed(visit_segments.reshape(-1), embedding_table.reshape(-1))
    p = p.reshape(_L, _D // 8, _NW, 8, _BPW)
    return p.transpose(2, 4, 0, 1, 3).reshape(_B, _L, _D)
